# Initial kernel scaffold; baseline (speedup 1.0000x reference)
import jax
import jax.numpy as jnp
from jax.experimental import pallas as pl


def kernel(q_v, q_a0, n_a0, q_a1, n_a1, q_a2, n_a2, query, pos, neg, emb, wh_w, wh_b, wp_w, wp_b):
    raise NotImplementedError("write your pallas kernel here")


# speedup vs baseline: 9.6596x; 9.6596x over previous
import functools

import jax
import jax.numpy as jnp
from jax import lax
from jax.experimental import pallas as pl
from jax.experimental.pallas import tpu as pltpu
from jax.experimental.pallas import tpu_sc as plsc

VOCAB = 1000000
D = 64
H = 50
O = 30
B = 4096
L = 50
DELTA = 1.0

NC = 2
NS = 16
NW = NC * NS

POOL_ARRAYS = 7
POOL_TASKS = POOL_ARRAYS * B // 2
POOL_TPW = POOL_TASKS // NW
POOL_CHUNK = 16
POOL_NCHUNK = POOL_TPW // POOL_CHUNK

MLP_ARRAYS = 3
BCHUNK = 128
MLP_TASKS = MLP_ARRAYS * L * (B // BCHUNK)
MLP_TPW = MLP_TASKS // NW
MLP_STAGE = 160

def _sc_body(emb, pool_idx, mlp_idx, pool_out, gath_out,
             idx_c, idx_m, buf_a, buf_b, mbuf_a, mbuf_b, outc,
             sem_a, sem_b):
  wid = lax.axis_index("s") * NC + lax.axis_index("c")

  def fire(idx_row, buf, sem):
    pltpu.make_async_copy(emb.at[idx_row], buf, sem).start()

  def wait(idx_row, buf, sem):
    pltpu.make_async_copy(emb.at[idx_row], buf, sem).wait()

  def accum50(buf, start, out_row):
    def body(i, accs):
      return tuple(accs[j] + buf[start + i, pl.ds(16 * j, 16)]
                   for j in range(4))
    z = jnp.zeros((16,), jnp.float32)
    a = lax.fori_loop(0, L, body, (z, z, z, z))
    for j in range(4):
      outc[out_row, pl.ds(16 * j, 16)] = a[j]

  def pool_chunk(c, _):
    tbase = wid * POOL_TPW + c * POOL_CHUNK
    pltpu.sync_copy(pool_idx.at[pl.ds(tbase, POOL_CHUNK)], idx_c)
    fire(idx_c.at[0], buf_a, sem_a)

    def pair(g, _):
      t0 = 2 * g
      t1 = t0 + 1
      fire(idx_c.at[t1], buf_b, sem_b)
      wait(idx_c.at[t0], buf_a, sem_a)
      accum50(buf_a, 0, 2 * t0)
      accum50(buf_a, L, 2 * t0 + 1)

      @pl.when(g < POOL_CHUNK // 2 - 1)
      def _():
        fire(idx_c.at[t0 + 2], buf_a, sem_a)

      wait(idx_c.at[t1], buf_b, sem_b)
      accum50(buf_b, 0, 2 * t1)
      accum50(buf_b, L, 2 * t1 + 1)
      return 0

    lax.fori_loop(0, POOL_CHUNK // 2, pair, 0)
    pltpu.sync_copy(outc, pool_out.at[pl.ds(2 * tbase, 2 * POOL_CHUNK)])
    return 0

  lax.fori_loop(0, POOL_NCHUNK, pool_chunk, 0)

  mbase = wid * MLP_TPW
  mstart = mbase // 8 * 8
  off = mbase - mstart
  pltpu.sync_copy(mlp_idx.at[pl.ds(mstart, MLP_STAGE)], idx_m)
  fire(idx_m.at[off], mbuf_a, sem_a)

  def mpair(g, _):
    t0 = 2 * g
    t1 = t0 + 1
    fire(idx_m.at[off + t1], mbuf_b, sem_b)
    wait(idx_m.at[off + t0], mbuf_a, sem_a)
    pltpu.sync_copy(mbuf_a, gath_out.at[pl.ds(BCHUNK * (mbase + t0), BCHUNK)])

    @pl.when(t0 + 2 < MLP_TPW)
    def _():
      fire(idx_m.at[off + t0 + 2], mbuf_a, sem_a)

    wait(idx_m.at[off + t1], mbuf_b, sem_b)
    pltpu.sync_copy(mbuf_b, gath_out.at[pl.ds(BCHUNK * (mbase + t1), BCHUNK)])
    return 0

  lax.fori_loop(0, MLP_TPW // 2, mpair, 0)


@functools.cache
def _sc_gather():
  mesh = plsc.VectorSubcoreMesh(core_axis_name="c", subcore_axis_name="s")
  return pl.kernel(
      _sc_body,
      mesh=mesh,
      compiler_params=pltpu.CompilerParams(use_tc_tiling_on_sc=False),
      out_type=[
          jax.ShapeDtypeStruct((POOL_ARRAYS * B, D), jnp.float32),
          jax.ShapeDtypeStruct((MLP_ARRAYS * L * B, D), jnp.float32),
      ],
      scratch_types=[
          pltpu.VMEM((POOL_CHUNK, 2 * L), jnp.int32),
          pltpu.VMEM((MLP_STAGE, BCHUNK), jnp.int32),
          pltpu.VMEM((2 * L, D), jnp.float32),
          pltpu.VMEM((2 * L, D), jnp.float32),
          pltpu.VMEM((BCHUNK, D), jnp.float32),
          pltpu.VMEM((BCHUNK, D), jnp.float32),
          pltpu.VMEM((2 * POOL_CHUNK, D), jnp.float32),
          pltpu.SemaphoreType.DMA,
          pltpu.SemaphoreType.DMA,
      ],
  )


def _tc_body(pooled_ref, gath_ref, whw_ref, whb_ref, wpw_ref, wpb_ref,
             out_ref):
  inv = 1.0 / (L * L)
  qv = pooled_ref[0]
  total = jnp.zeros((BCHUNK,), jnp.float32)
  for k in range(3):
    dq = jnp.sum(qv * pooled_ref[1 + 2 * k], axis=1)
    dn = jnp.sum(qv * pooled_ref[2 + 2 * k], axis=1)
    total = total + jnp.maximum(DELTA - inv * dq + inv * dn, 0.0)

  whw = whw_ref[...]
  whb = whb_ref[...]
  wpw = wpw_ref[...]
  wpb = wpb_ref[...]

  def proj(e):
    z = lax.dot_general(e, whw, (((1,), (1,)), ((), ())),
                        preferred_element_type=jnp.float32) + whb
    h = 1.0 / (1.0 + jnp.exp(-z))
    return lax.dot_general(h, wpw, (((1,), (1,)), ((), ())),
                           preferred_element_type=jnp.float32) + wpb

  oq = proj(gath_ref[0].reshape(L * BCHUNK, D))
  op_ = proj(gath_ref[1].reshape(L * BCHUNK, D))
  on_ = proj(gath_ref[2].reshape(L * BCHUNK, D))
  dqp = jnp.sum(jnp.sum(oq * op_, axis=1).reshape(L, BCHUNK), axis=0)
  dqn = jnp.sum(jnp.sum(oq * on_, axis=1).reshape(L, BCHUNK), axis=0)
  out_ref[0, 0, :] = total + jnp.maximum(DELTA - dqp + dqn, 0.0)


def _tc_call(pooled3, gath4, wh_w, wh_b2, wp_w, wp_b2):
  nblk = B // BCHUNK
  return pl.pallas_call(
      _tc_body,
      grid=(nblk,),
      in_specs=[
          pl.BlockSpec((POOL_ARRAYS, BCHUNK, D), lambda i: (0, i, 0)),
          pl.BlockSpec((MLP_ARRAYS, L, BCHUNK, D), lambda i: (0, 0, i, 0)),
          pl.BlockSpec((H, D), lambda i: (0, 0)),
          pl.BlockSpec((1, H), lambda i: (0, 0)),
          pl.BlockSpec((O, H), lambda i: (0, 0)),
          pl.BlockSpec((1, O), lambda i: (0, 0)),
      ],
      out_specs=pl.BlockSpec((1, 1, BCHUNK), lambda i: (i, 0, 0)),
      out_shape=jax.ShapeDtypeStruct((nblk, 1, BCHUNK), jnp.float32),
  )(pooled3, gath4, wh_w, wh_b2, wp_w, wp_b2)


def kernel(q_v, q_a0, n_a0, q_a1, n_a1, q_a2, n_a2, query, pos, neg,
           emb, wh_w, wh_b, wp_w, wp_b):
  pool_idx = jnp.concatenate(
      [q_v, q_a0, n_a0, q_a1, n_a1, q_a2, n_a2], axis=0
  ).astype(jnp.int32).reshape(POOL_TASKS, 2 * L)
  mlp_idx = (jnp.stack([query, pos, neg], axis=0).astype(jnp.int32)
             .transpose(0, 2, 1).reshape(MLP_TASKS, BCHUNK))
  mlp_idx = jnp.pad(mlp_idx, ((0, MLP_STAGE), (0, 0)))

  pooled, gath = _sc_gather()(emb, pool_idx, mlp_idx)
  pooled3 = pooled.reshape(POOL_ARRAYS, B, D)
  gath4 = gath.reshape(MLP_ARRAYS, L, B, D)
  out = _tc_call(pooled3, gath4, wh_w, wh_b.reshape(1, H),
                 wp_w, wp_b.reshape(1, O))
  return out.reshape(B)
